# baseline (speedup 1.0000x reference)
import functools

import jax
import jax.numpy as jnp
from jax import lax
from jax.experimental import pallas as pl
from jax.experimental.pallas import tpu as pltpu
from jax.experimental.pallas import tpu_sc as plsc

VOCAB_SIZE = 1000000
EMBED_DIM = 32
BATCH = 16384
N_FIELDS = 26

NUM_CORES = 2
NUM_SUBCORES = 16
NUM_WORKERS = NUM_CORES * NUM_SUBCORES

TOTAL_ROWS = BATCH * N_FIELDS
ROWS_PER_WORKER = TOTAL_ROWS // NUM_WORKERS
CHUNK = 1664
N_CHUNKS = ROWS_PER_WORKER // CHUNK

assert ROWS_PER_WORKER * NUM_WORKERS == TOTAL_ROWS
assert N_CHUNKS * CHUNK == ROWS_PER_WORKER and N_CHUNKS >= 2

_mesh = plsc.VectorSubcoreMesh(
    core_axis_name="c", subcore_axis_name="s",
    num_cores=NUM_CORES, num_subcores=NUM_SUBCORES,
)


@functools.partial(
    pl.kernel,
    mesh=_mesh,
    compiler_params=pltpu.CompilerParams(use_tc_tiling_on_sc=False),
    out_type=jax.ShapeDtypeStruct((TOTAL_ROWS, EMBED_DIM), jnp.float32),
    scratch_types=[
        pltpu.VMEM((ROWS_PER_WORKER,), jnp.int32),
        pltpu.VMEM((CHUNK, EMBED_DIM), jnp.float32),
        pltpu.VMEM((CHUNK, EMBED_DIM), jnp.float32),
        pltpu.SemaphoreType.DMA,
        pltpu.SemaphoreType.DMA,
        pltpu.SemaphoreType.DMA,
        pltpu.SemaphoreType.DMA,
    ],
)
def _gather_kernel(idx_hbm, table_hbm, out_hbm, idx_v, rows0, rows1,
                   gsem0, gsem1, osem0, osem1):
    wid = lax.axis_index("s") * NUM_CORES + lax.axis_index("c")
    base = wid * ROWS_PER_WORKER

    rows = (rows0, rows1)
    gsems = (gsem0, gsem1)
    osems = (osem0, osem1)

    pltpu.sync_copy(idx_hbm.at[pl.ds(base, ROWS_PER_WORKER)], idx_v)

    def gather(g):
        b = g & 1
        return pltpu.async_copy(
            table_hbm.at[idx_v.at[pl.ds(g * CHUNK, CHUNK)]], rows[b], gsems[b])

    gathers = [None] * N_CHUNKS
    writebacks = [None] * N_CHUNKS
    gathers[0] = gather(0)
    for g in range(N_CHUNKS):
        b = g & 1
        if g + 1 < N_CHUNKS:
            if g >= 1:
                writebacks[g - 1].wait()
            gathers[g + 1] = gather(g + 1)
        gathers[g].wait()
        writebacks[g] = pltpu.async_copy(
            rows[b], out_hbm.at[pl.ds(base + g * CHUNK, CHUNK)], osems[b])
    writebacks[N_CHUNKS - 2].wait()
    writebacks[N_CHUNKS - 1].wait()


def kernel(ability_name, ability_embed_weight):
    flat_idx = ability_name.reshape(TOTAL_ROWS)
    out = _gather_kernel(flat_idx, ability_embed_weight)
    return out.reshape(BATCH, N_FIELDS, EMBED_DIM)
